# baseline (speedup 1.0000x reference)
import functools

import jax
import jax.numpy as jnp
from jax import lax
from jax.experimental import pallas as pl
from jax.experimental.pallas import tpu as pltpu
from jax.experimental.pallas import tpu_sc as plsc

_N_USERS = 25000
_N_NODES = 50000
_D = 64
_E = 800000
_BATCH = 4096

_NT = 16
_CH = 96
_NB = 8
_DG = 5
_EP = 811008
_CHUNKS = _EP // (_NT * _CH)
_NP = 50176
_SLAB = _NP // _NT


def _segsum(ego2, colg, rowg, valg):
  mesh = plsc.VectorSubcoreMesh(core_axis_name="c", subcore_axis_name="s",
                                num_cores=2, num_subcores=_NT)

  @functools.partial(
      pl.kernel,
      out_type=jax.ShapeDtypeStruct((2, _NP, 32), jnp.float32),
      mesh=mesh,
      scratch_types=[
          [pltpu.VMEM((1, _CH), jnp.int32)] * _NB,
          [pltpu.VMEM((1, _CH), jnp.int32)] * _NB,
          [pltpu.VMEM((_CH + 16,), jnp.float32)] * _NB,
          [pltpu.VMEM((_CH, 32), jnp.float32)] * _NB,
          pltpu.VMEM_SHARED((_NP, 32), jnp.float32),
          [pltpu.SemaphoreType.DMA] * _NB,
          [pltpu.SemaphoreType.DMA] * _NB,
          [pltpu.SemaphoreType.DMA] * _NB,
      ],
      compiler_params=pltpu.CompilerParams(use_tc_tiling_on_sc=False),
  )
  def k(ego_hbm, col_hbm, row_hbm, val_hbm, out_hbm,
        colv, dstv, valv, rows, acc, stsem, gsem, ssem):
    c = lax.axis_index("c")
    s = lax.axis_index("s")

    for r in range(_NB):
      @plsc.parallel_loop(0, _CH, unroll=8)
      def _(t, r=r):
        z = jnp.zeros((16,), jnp.float32)
        rows[r][t, pl.ds(0, 16)] = z
        rows[r][t, pl.ds(16, 16)] = z

    zcps = []
    for t in range(32):
      zcps.append(pltpu.async_copy(
          rows[t % _NB].at[pl.ds(0, _CH)],
          acc.at[pl.ds(s * _SLAB + t * _CH, _CH)], stsem[t % _NB]))
    zcps.append(pltpu.async_copy(
        rows[0].at[pl.ds(0, 64)],
        acc.at[pl.ds(s * _SLAB + 3072, 64)], stsem[0]))
    for cp in zcps:
      cp.wait()
    plsc.subcore_barrier()

    def stage_start(sj, n):
      rn = s * _CHUNKS + n
      pltpu.async_copy(col_hbm.at[pl.ds(rn, 1)], colv[sj], stsem[sj])
      pltpu.async_copy(row_hbm.at[pl.ds(rn, 1)], dstv[sj], stsem[sj])
      pltpu.async_copy(val_hbm.at[pl.ds(rn * _CH, _CH)],
                       valv[sj].at[pl.ds(0, _CH)], stsem[sj])

    def stage_wait(sj, n):
      rn = s * _CHUNKS + n
      pltpu.make_async_copy(col_hbm.at[pl.ds(rn, 1)], colv[sj],
                            stsem[sj]).wait()
      pltpu.make_async_copy(row_hbm.at[pl.ds(rn, 1)], dstv[sj],
                            stsem[sj]).wait()
      pltpu.make_async_copy(val_hbm.at[pl.ds(rn * _CH, _CH)],
                            valv[sj].at[pl.ds(0, _CH)], stsem[sj]).wait()

    def fix_idx(sj):
      for t in range(_CH // 16):
        sl = pl.ds(t * 16, 16)
        colv[sj][0, sl] = colv[sj][0, sl] * 2 + c

    def gather_start(sj):
      pltpu.async_copy(ego_hbm.at[colv[sj].at[0]], rows[sj], gsem[sj])

    def gather_wait(sj):
      pltpu.make_async_copy(ego_hbm.at[colv[sj].at[0]], rows[sj],
                            gsem[sj]).wait()

    def scatter_start(sj):
      pltpu.async_copy(rows[sj], acc.at[dstv[sj].at[0]], ssem[sj], add=True)

    def scatter_wait(sj):
      pltpu.make_async_copy(rows[sj], acc.at[dstv[sj].at[0]],
                            ssem[sj]).wait()

    def scale(sj):
      @plsc.parallel_loop(0, _CH, unroll=8)
      def _(e):
        v = valv[sj][pl.ds(e, 16)][0]
        rows[sj][e, pl.ds(0, 16)] = rows[sj][e, pl.ds(0, 16)] * v
        rows[sj][e, pl.ds(16, 16)] = rows[sj][e, pl.ds(16, 16)] * v

    zero = jnp.int32(0)
    for m in range(_DG + 1):
      stage_start(m, zero + m)
    for m in range(_DG):
      stage_wait(m, zero + m)
      fix_idx(m)
      gather_start(m)

    def body(qq, carry):
      n0 = qq * _NB
      last = qq == _CHUNKS // _NB - 1
      for j in range(_NB):
        n = n0 + j
        sg = (j + _DG) % _NB
        ss = (j + _DG + 1) % _NB

        gather_wait(j)

        if j >= 2:
          scatter_wait(j - 2)
        else:
          @pl.when(qq > 0)
          def _():
            scatter_wait((j - 2) % _NB)

        def prep_next():
          stage_wait(sg, n + _DG)
          fix_idx(sg)
          gather_start(sg)

        if j + _DG < _NB:
          prep_next()
        else:
          @pl.when(jnp.logical_not(last))
          def _():
            prep_next()

        scale(j)
        scatter_start(j)

        if j + _DG + 1 < _NB:
          stage_start(ss, n + _DG + 1)
        else:
          @pl.when(jnp.logical_not(last))
          def _():
            stage_start(ss, n + _DG + 1)
      return carry

    lax.fori_loop(0, _CHUNKS // _NB, body, 0)
    scatter_wait(_NB - 2)
    scatter_wait(_NB - 1)
    plsc.subcore_barrier()

    for t in range(6):
      sl = pl.ds(s * _SLAB + t * 512, 512)
      pltpu.sync_copy(acc.at[sl], out_hbm.at[c].at[sl])
    sl = pl.ds(s * _SLAB + 3072, 64)
    pltpu.sync_copy(acc.at[sl], out_hbm.at[c].at[sl])

  return k(ego2, colg, rowg, valg)


def _transform(side, ego, W_gc, b_gc, W_bi, b_bi):
  R = 2000

  def body(s_ref, e_ref, wg_ref, bg_ref, wb_ref, bb_ref, eout_ref, nout_ref):
    s_lo = s_ref[0]
    s_hi = s_ref[1]
    e = e_ref[...]
    e_lo = e[:, :32]
    e_hi = e[:, 32:]
    wg = wg_ref[...]
    wb = wb_ref[...]
    sum_e = (jnp.dot(s_lo, wg[:32], preferred_element_type=jnp.float32) +
             jnp.dot(s_hi, wg[32:], preferred_element_type=jnp.float32) +
             bg_ref[...])
    bi_e = (jnp.dot(e_lo * s_lo, wb[:32], preferred_element_type=jnp.float32) +
            jnp.dot(e_hi * s_hi, wb[32:], preferred_element_type=jnp.float32) +
            bb_ref[...])
    act = sum_e + bi_e
    act = jnp.where(act >= 0, act, 0.2 * act)
    eout_ref[...] = act
    ss = jnp.sum(act * act, axis=1, keepdims=True)
    nout_ref[...] = act / jnp.maximum(jnp.sqrt(ss), 1e-12)

  return pl.pallas_call(
      body,
      grid=(_N_NODES // R,),
      in_specs=[
          pl.BlockSpec((2, R, 32), lambda r: (0, r, 0)),
          pl.BlockSpec((R, _D), lambda r: (r, 0)),
          pl.BlockSpec((_D, _D), lambda r: (0, 0)),
          pl.BlockSpec((1, _D), lambda r: (0, 0)),
          pl.BlockSpec((_D, _D), lambda r: (0, 0)),
          pl.BlockSpec((1, _D), lambda r: (0, 0)),
      ],
      out_specs=[
          pl.BlockSpec((R, _D), lambda r: (r, 0)),
          pl.BlockSpec((R, _D), lambda r: (r, 0)),
      ],
      out_shape=[jax.ShapeDtypeStruct((_N_NODES, _D), jnp.float32)] * 2,
      compiler_params=pltpu.CompilerParams(
          dimension_semantics=("arbitrary",)),
  )(side, ego, W_gc, b_gc, W_bi, b_bi)


def _final(t0, t1, t2, t3, u2, i2, j2):
  mesh = plsc.VectorSubcoreMesh(core_axis_name="c", subcore_axis_name="s",
                                num_cores=2, num_subcores=_NT)
  out_sd = jax.ShapeDtypeStruct((_BATCH, _D), jnp.float32)

  @functools.partial(
      pl.kernel,
      out_type=[out_sd] * 12,
      mesh=mesh,
      scratch_types=[
          pltpu.VMEM((128,), jnp.int32),
          pltpu.VMEM((128, _D), jnp.float32),
          pltpu.SemaphoreType.DMA,
      ],
      compiler_params=pltpu.CompilerParams(use_tc_tiling_on_sc=False),
  )
  def k(t0_hbm, t1_hbm, t2_hbm, t3_hbm, u_hbm, i_hbm, j_hbm, *rest):
    outs = rest[:12]
    idxv, buf, sem = rest[12:]
    c = lax.axis_index("c")
    s = lax.axis_index("s")
    w = s * 2 + c
    for p, (src, off) in enumerate(((u_hbm, 0), (i_hbm, _N_USERS),
                                    (j_hbm, _N_USERS))):
      pltpu.sync_copy(src.at[w], idxv)
      if off:
        for t in range(8):
          sl = pl.ds(t * 16, 16)
          idxv[sl] = idxv[sl] + off
      for tt, tbl in enumerate((t0_hbm, t1_hbm, t2_hbm, t3_hbm)):
        pltpu.async_copy(tbl.at[idxv], buf, sem).wait()
        pltpu.sync_copy(buf, outs[p * 4 + tt].at[pl.ds(w * 128, 128)])

  return k(t0, t1, t2, t3, u2, i2, j2)


def kernel(u, i, j, adj_indices, adj_values, user_emb, item_emb,
           W_gc_0, b_gc_0, W_bi_0, b_bi_0,
           W_gc_1, b_gc_1, W_bi_1, b_bi_1,
           W_gc_2, b_gc_2, W_bi_2, b_bi_2):
  rows_d = adj_indices[0].astype(jnp.int32)
  cols = adj_indices[1].astype(jnp.int32)
  vals = adj_values
  pad = _EP - _E
  colg = jnp.concatenate([cols, jnp.zeros((pad,), jnp.int32)]).reshape(
      _EP // _CH, _CH)
  rowg = jnp.concatenate([rows_d, jnp.zeros((pad,), jnp.int32)]).reshape(
      _EP // _CH, _CH)
  valg = jnp.concatenate([vals, jnp.zeros((pad,), jnp.float32)])

  ego = jnp.concatenate([user_emb, item_emb], axis=0)
  tables = [ego]
  for (Wg, bg, Wb, bb) in ((W_gc_0, b_gc_0, W_bi_0, b_bi_0),
                           (W_gc_1, b_gc_1, W_bi_1, b_bi_1),
                           (W_gc_2, b_gc_2, W_bi_2, b_bi_2)):
    side = _segsum(ego.reshape(2 * _N_NODES, 32), colg, rowg, valg)
    ego, nk = _transform(side, ego, Wg, bg, Wb, bb)
    tables.append(nk)

  u2 = u.astype(jnp.int32).reshape(32, 128)
  i2 = i.astype(jnp.int32).reshape(32, 128)
  j2 = j.astype(jnp.int32).reshape(32, 128)
  parts = _final(tables[0], tables[1], tables[2], tables[3], u2, i2, j2)
  return (jnp.concatenate(parts[0:4], axis=1),
          jnp.concatenate(parts[4:8], axis=1),
          jnp.concatenate(parts[8:12], axis=1))


# speedup vs baseline: 1.0615x; 1.0615x over previous
import functools

import jax
import jax.numpy as jnp
from jax import lax
from jax.experimental import pallas as pl
from jax.experimental.pallas import tpu as pltpu
from jax.experimental.pallas import tpu_sc as plsc

_N_USERS = 25000
_N_NODES = 50000
_D = 64
_E = 800000
_BATCH = 4096

_NT = 16
_CH = 128
_NB = 6
_DG = 3
_EP = 811008
_CHUNKS = _EP // (_NT * _CH)
_NP = 50176
_SLAB = _NP // _NT


def _segsum(ego2, colg, rowg, valg):
  mesh = plsc.VectorSubcoreMesh(core_axis_name="c", subcore_axis_name="s",
                                num_cores=2, num_subcores=_NT)

  @functools.partial(
      pl.kernel,
      out_type=jax.ShapeDtypeStruct((2, _NP, 32), jnp.float32),
      mesh=mesh,
      scratch_types=[
          [pltpu.VMEM((1, _CH), jnp.int32)] * _NB,
          [pltpu.VMEM((1, _CH), jnp.int32)] * _NB,
          [pltpu.VMEM((_CH + 16,), jnp.float32)] * _NB,
          [pltpu.VMEM((_CH, 32), jnp.float32)] * _NB,
          pltpu.VMEM_SHARED((_NP, 32), jnp.float32),
          [pltpu.SemaphoreType.DMA] * _NB,
          [pltpu.SemaphoreType.DMA] * _NB,
          [pltpu.SemaphoreType.DMA] * _NB,
      ],
      compiler_params=pltpu.CompilerParams(use_tc_tiling_on_sc=False),
  )
  def k(ego_hbm, col_hbm, row_hbm, val_hbm, out_hbm,
        colv, dstv, valv, rows, acc, stsem, gsem, ssem):
    c = lax.axis_index("c")
    s = lax.axis_index("s")

    for r in range(_NB):
      @plsc.parallel_loop(0, _CH, unroll=8)
      def _(t, r=r):
        z = jnp.zeros((16,), jnp.float32)
        rows[r][t, pl.ds(0, 16)] = z
        rows[r][t, pl.ds(16, 16)] = z

    zcps = []
    for t in range(3072 // _CH):
      zcps.append(pltpu.async_copy(
          rows[t % _NB].at[pl.ds(0, _CH)],
          acc.at[pl.ds(s * _SLAB + t * _CH, _CH)], stsem[t % _NB]))
    zcps.append(pltpu.async_copy(
        rows[0].at[pl.ds(0, 64)],
        acc.at[pl.ds(s * _SLAB + 3072, 64)], stsem[0]))
    for cp in zcps:
      cp.wait()
    plsc.subcore_barrier()

    def stage_start(sj, n):
      rn = s * _CHUNKS + n
      pltpu.async_copy(col_hbm.at[pl.ds(rn, 1)], colv[sj], stsem[sj])
      pltpu.async_copy(row_hbm.at[pl.ds(rn, 1)], dstv[sj], stsem[sj])
      pltpu.async_copy(val_hbm.at[pl.ds(rn * _CH, _CH)],
                       valv[sj].at[pl.ds(0, _CH)], stsem[sj])

    def stage_wait(sj, n):
      rn = s * _CHUNKS + n
      pltpu.make_async_copy(col_hbm.at[pl.ds(rn, 1)], colv[sj],
                            stsem[sj]).wait()
      pltpu.make_async_copy(row_hbm.at[pl.ds(rn, 1)], dstv[sj],
                            stsem[sj]).wait()
      pltpu.make_async_copy(val_hbm.at[pl.ds(rn * _CH, _CH)],
                            valv[sj].at[pl.ds(0, _CH)], stsem[sj]).wait()

    def fix_idx(sj):
      for t in range(_CH // 16):
        sl = pl.ds(t * 16, 16)
        colv[sj][0, sl] = colv[sj][0, sl] * 2 + c

    def gather_start(sj):
      pltpu.async_copy(ego_hbm.at[colv[sj].at[0]], rows[sj], gsem[sj])

    def gather_wait(sj):
      pltpu.make_async_copy(ego_hbm.at[colv[sj].at[0]], rows[sj],
                            gsem[sj]).wait()

    def scatter_start(sj):
      pltpu.async_copy(rows[sj], acc.at[dstv[sj].at[0]], ssem[sj], add=True)

    def scatter_wait(sj):
      pltpu.make_async_copy(rows[sj], acc.at[dstv[sj].at[0]],
                            ssem[sj]).wait()

    def scale(sj):
      @plsc.parallel_loop(0, _CH, unroll=8)
      def _(e):
        v = valv[sj][pl.ds(e, 16)][0]
        rows[sj][e, pl.ds(0, 16)] = rows[sj][e, pl.ds(0, 16)] * v
        rows[sj][e, pl.ds(16, 16)] = rows[sj][e, pl.ds(16, 16)] * v

    zero = jnp.int32(0)
    for m in range(_DG + 1):
      stage_start(m, zero + m)
    for m in range(_DG):
      stage_wait(m, zero + m)
      fix_idx(m)
      gather_start(m)

    def body(qq, carry):
      n0 = qq * _NB
      last = qq == _CHUNKS // _NB - 1
      for j in range(_NB):
        n = n0 + j
        sg = (j + _DG) % _NB
        ss = (j + _DG + 1) % _NB

        gather_wait(j)

        if j >= 2:
          scatter_wait(j - 2)
        else:
          @pl.when(qq > 0)
          def _():
            scatter_wait((j - 2) % _NB)

        def prep_next():
          stage_wait(sg, n + _DG)
          fix_idx(sg)
          gather_start(sg)

        if j + _DG < _NB:
          prep_next()
        else:
          @pl.when(jnp.logical_not(last))
          def _():
            prep_next()

        scale(j)
        scatter_start(j)

        if j + _DG + 1 < _NB:
          stage_start(ss, n + _DG + 1)
        else:
          @pl.when(jnp.logical_not(last))
          def _():
            stage_start(ss, n + _DG + 1)
      return carry

    lax.fori_loop(0, _CHUNKS // _NB, body, 0)
    scatter_wait(_NB - 2)
    scatter_wait(_NB - 1)
    plsc.subcore_barrier()

    for t in range(6):
      sl = pl.ds(s * _SLAB + t * 512, 512)
      pltpu.sync_copy(acc.at[sl], out_hbm.at[c].at[sl])
    sl = pl.ds(s * _SLAB + 3072, 64)
    pltpu.sync_copy(acc.at[sl], out_hbm.at[c].at[sl])

  return k(ego2, colg, rowg, valg)


def _transform(side, ego, W_gc, b_gc, W_bi, b_bi):
  R = 2000

  def body(s_ref, e_ref, wg_ref, bg_ref, wb_ref, bb_ref, eout_ref, nout_ref):
    s_lo = s_ref[0]
    s_hi = s_ref[1]
    e = e_ref[...]
    e_lo = e[:, :32]
    e_hi = e[:, 32:]
    wg = wg_ref[...]
    wb = wb_ref[...]
    sum_e = (jnp.dot(s_lo, wg[:32], preferred_element_type=jnp.float32) +
             jnp.dot(s_hi, wg[32:], preferred_element_type=jnp.float32) +
             bg_ref[...])
    bi_e = (jnp.dot(e_lo * s_lo, wb[:32], preferred_element_type=jnp.float32) +
            jnp.dot(e_hi * s_hi, wb[32:], preferred_element_type=jnp.float32) +
            bb_ref[...])
    act = sum_e + bi_e
    act = jnp.where(act >= 0, act, 0.2 * act)
    eout_ref[...] = act
    ss = jnp.sum(act * act, axis=1, keepdims=True)
    nout_ref[...] = act / jnp.maximum(jnp.sqrt(ss), 1e-12)

  return pl.pallas_call(
      body,
      grid=(_N_NODES // R,),
      in_specs=[
          pl.BlockSpec((2, R, 32), lambda r: (0, r, 0)),
          pl.BlockSpec((R, _D), lambda r: (r, 0)),
          pl.BlockSpec((_D, _D), lambda r: (0, 0)),
          pl.BlockSpec((1, _D), lambda r: (0, 0)),
          pl.BlockSpec((_D, _D), lambda r: (0, 0)),
          pl.BlockSpec((1, _D), lambda r: (0, 0)),
      ],
      out_specs=[
          pl.BlockSpec((R, _D), lambda r: (r, 0)),
          pl.BlockSpec((R, _D), lambda r: (r, 0)),
      ],
      out_shape=[jax.ShapeDtypeStruct((_N_NODES, _D), jnp.float32)] * 2,
      compiler_params=pltpu.CompilerParams(
          dimension_semantics=("arbitrary",)),
  )(side, ego, W_gc, b_gc, W_bi, b_bi)


def _final(t0, t1, t2, t3, u2, i2, j2):
  mesh = plsc.VectorSubcoreMesh(core_axis_name="c", subcore_axis_name="s",
                                num_cores=2, num_subcores=_NT)
  out_sd = jax.ShapeDtypeStruct((_BATCH, _D), jnp.float32)

  @functools.partial(
      pl.kernel,
      out_type=[out_sd] * 12,
      mesh=mesh,
      scratch_types=[
          pltpu.VMEM((128,), jnp.int32),
          pltpu.VMEM((128, _D), jnp.float32),
          pltpu.SemaphoreType.DMA,
      ],
      compiler_params=pltpu.CompilerParams(use_tc_tiling_on_sc=False),
  )
  def k(t0_hbm, t1_hbm, t2_hbm, t3_hbm, u_hbm, i_hbm, j_hbm, *rest):
    outs = rest[:12]
    idxv, buf, sem = rest[12:]
    c = lax.axis_index("c")
    s = lax.axis_index("s")
    w = s * 2 + c
    for p, (src, off) in enumerate(((u_hbm, 0), (i_hbm, _N_USERS),
                                    (j_hbm, _N_USERS))):
      pltpu.sync_copy(src.at[w], idxv)
      if off:
        for t in range(8):
          sl = pl.ds(t * 16, 16)
          idxv[sl] = idxv[sl] + off
      for tt, tbl in enumerate((t0_hbm, t1_hbm, t2_hbm, t3_hbm)):
        pltpu.async_copy(tbl.at[idxv], buf, sem).wait()
        pltpu.sync_copy(buf, outs[p * 4 + tt].at[pl.ds(w * 128, 128)])

  return k(t0, t1, t2, t3, u2, i2, j2)


def kernel(u, i, j, adj_indices, adj_values, user_emb, item_emb,
           W_gc_0, b_gc_0, W_bi_0, b_bi_0,
           W_gc_1, b_gc_1, W_bi_1, b_bi_1,
           W_gc_2, b_gc_2, W_bi_2, b_bi_2):
  rows_d = adj_indices[0].astype(jnp.int32)
  cols = adj_indices[1].astype(jnp.int32)
  vals = adj_values
  pad = _EP - _E
  colg = jnp.concatenate([cols, jnp.zeros((pad,), jnp.int32)]).reshape(
      _EP // _CH, _CH)
  rowg = jnp.concatenate([rows_d, jnp.zeros((pad,), jnp.int32)]).reshape(
      _EP // _CH, _CH)
  valg = jnp.concatenate([vals, jnp.zeros((pad,), jnp.float32)])

  ego = jnp.concatenate([user_emb, item_emb], axis=0)
  tables = [ego]
  for (Wg, bg, Wb, bb) in ((W_gc_0, b_gc_0, W_bi_0, b_bi_0),
                           (W_gc_1, b_gc_1, W_bi_1, b_bi_1),
                           (W_gc_2, b_gc_2, W_bi_2, b_bi_2)):
    side = _segsum(ego.reshape(2 * _N_NODES, 32), colg, rowg, valg)
    ego, nk = _transform(side, ego, Wg, bg, Wb, bb)
    tables.append(nk)

  u2 = u.astype(jnp.int32).reshape(32, 128)
  i2 = i.astype(jnp.int32).reshape(32, 128)
  j2 = j.astype(jnp.int32).reshape(32, 128)
  parts = _final(tables[0], tables[1], tables[2], tables[3], u2, i2, j2)
  return (jnp.concatenate(parts[0:4], axis=1),
          jnp.concatenate(parts[4:8], axis=1),
          jnp.concatenate(parts[8:12], axis=1))


# speedup vs baseline: 1.5783x; 1.4868x over previous
import functools

import jax
import jax.numpy as jnp
from jax import lax
from jax.experimental import pallas as pl
from jax.experimental.pallas import tpu as pltpu
from jax.experimental.pallas import tpu_sc as plsc

_N_USERS = 25000
_N_NODES = 50000
_D = 64
_E = 800000
_BATCH = 4096

_NT = 16
_CH = 256
_G = _CH // 128
_EP = 802816
_CHUNKS = _EP // (_NT * _CH)
_NP = 50176
_SLAB = _NP // _NT


def _segsum(ego2, colg, rowg, valg):
  mesh = plsc.VectorSubcoreMesh(core_axis_name="c", subcore_axis_name="s",
                                num_cores=2, num_subcores=_NT)

  @functools.partial(
      pl.kernel,
      out_type=jax.ShapeDtypeStruct((2, _NP, 32), jnp.float32),
      mesh=mesh,
      scratch_types=[
          [pltpu.VMEM((_G * 2, 64), jnp.int32)] * 4,
          [pltpu.VMEM((_G, 128), jnp.int32)] * 4,
          [pltpu.VMEM((_CH + 16,), jnp.float32)] * 4,
          [pltpu.VMEM((_CH, 32), jnp.float32)] * 2,
          pltpu.VMEM_SHARED((_NP, 32), jnp.float32),
          [pltpu.SemaphoreType.DMA] * 4,
          [pltpu.SemaphoreType.DMA] * 2,
          [pltpu.SemaphoreType.DMA] * 2,
      ],
      compiler_params=pltpu.CompilerParams(use_tc_tiling_on_sc=False),
  )
  def k(ego_hbm, col_hbm, row_hbm, val_hbm, out_hbm,
        colv, dstv, valv, rows, acc, stsem, gsem, ssem):
    c = lax.axis_index("c")
    s = lax.axis_index("s")

    for r in range(2):
      @plsc.parallel_loop(0, _CH, unroll=8)
      def _(t, r=r):
        z = jnp.zeros((16,), jnp.float32)
        rows[r][t, pl.ds(0, 16)] = z
        rows[r][t, pl.ds(16, 16)] = z

    zcps = []
    for t in range(12):
      zcps.append(pltpu.async_copy(
          rows[t % 2].at[pl.ds(0, 256)],
          acc.at[pl.ds(s * _SLAB + t * 256, 256)], stsem[t % 4]))
    zcps.append(pltpu.async_copy(
        rows[0].at[pl.ds(0, 64)],
        acc.at[pl.ds(s * _SLAB + 3072, 64)], stsem[0]))
    for cp in zcps:
      cp.wait()
    plsc.subcore_barrier()

    def stage_start(sj, n):
      row0 = (s * _CHUNKS + n) * _G
      e0 = (s * _CHUNKS + n) * _CH
      pltpu.async_copy(col_hbm.at[pl.ds(row0 * 2, _G * 2)], colv[sj],
                       stsem[sj])
      pltpu.async_copy(row_hbm.at[pl.ds(row0, _G)], dstv[sj], stsem[sj])
      pltpu.async_copy(val_hbm.at[pl.ds(e0, _CH)],
                       valv[sj].at[pl.ds(0, _CH)], stsem[sj])

    def stage_wait(sj, n):
      row0 = (s * _CHUNKS + n) * _G
      e0 = (s * _CHUNKS + n) * _CH
      pltpu.make_async_copy(col_hbm.at[pl.ds(row0 * 2, _G * 2)], colv[sj],
                            stsem[sj]).wait()
      pltpu.make_async_copy(row_hbm.at[pl.ds(row0, _G)], dstv[sj],
                            stsem[sj]).wait()
      pltpu.make_async_copy(val_hbm.at[pl.ds(e0, _CH)],
                            valv[sj].at[pl.ds(0, _CH)], stsem[sj]).wait()

    def fix_idx(sj):
      for g in range(_G * 2):
        for t in range(4):
          sl = pl.ds(t * 16, 16)
          colv[sj][g, sl] = colv[sj][g, sl] * 2 + c

    def gather_start(sj, rj):
      for g in range(_G * 2):
        pltpu.async_copy(ego_hbm.at[colv[sj].at[g]],
                         rows[rj].at[pl.ds(g * 64, 64)], gsem[rj])

    def gather_wait(sj, rj):
      for g in range(_G * 2):
        pltpu.make_async_copy(ego_hbm.at[colv[sj].at[g]],
                              rows[rj].at[pl.ds(g * 64, 64)],
                              gsem[rj]).wait()

    def scatter_start(sj, rj):
      for g in range(_G):
        pltpu.async_copy(rows[rj].at[pl.ds(g * 128, 128)],
                         acc.at[dstv[sj].at[g]], ssem[rj], add=True)

    def scatter_wait(sj, rj):
      for g in range(_G):
        pltpu.make_async_copy(rows[rj].at[pl.ds(g * 128, 128)],
                              acc.at[dstv[sj].at[g]], ssem[rj]).wait()

    def scale(sj, rj):
      @plsc.parallel_loop(0, _CH, unroll=8)
      def _(e):
        v = valv[sj][pl.ds(e, 16)][0]
        rows[rj][e, pl.ds(0, 16)] = rows[rj][e, pl.ds(0, 16)] * v
        rows[rj][e, pl.ds(16, 16)] = rows[rj][e, pl.ds(16, 16)] * v

    zero = jnp.int32(0)
    stage_start(0, zero)
    stage_start(1, zero + 1)
    stage_wait(0, zero)
    fix_idx(0)
    gather_start(0, 0)

    def body(qq, carry):
      n0 = qq * 4
      last = qq == _CHUNKS // 4 - 1
      for j in range(4):
        sj, rj = j, j % 2
        sj1, rj1 = (j + 1) % 4, (j + 1) % 2
        n = n0 + j

        gather_wait(sj, rj)

        def prep_next():
          stage_wait(sj1, n + 1)
          fix_idx(sj1)
          gather_start(sj1, rj1)

        if j == 3:
          @pl.when(jnp.logical_not(last))
          def _():
            scatter_wait((j - 1) % 4, rj1)
            prep_next()
        else:
          if j == 0:
            @pl.when(qq > 0)
            def _():
              scatter_wait(3, rj1)
          else:
            scatter_wait(j - 1, rj1)
          prep_next()

        scale(sj, rj)
        scatter_start(sj, rj)

        if j < 2:
          stage_start((j + 2) % 4, n + 2)
        else:
          @pl.when(jnp.logical_not(last))
          def _():
            stage_start((j + 2) % 4, n + 2)
      return carry

    lax.fori_loop(0, _CHUNKS // 4, body, 0)
    scatter_wait(2, 0)
    scatter_wait(3, 1)
    plsc.subcore_barrier()

    for t in range(6):
      sl = pl.ds(s * _SLAB + t * 512, 512)
      pltpu.sync_copy(acc.at[sl], out_hbm.at[c].at[sl])
    sl = pl.ds(s * _SLAB + 3072, 64)
    pltpu.sync_copy(acc.at[sl], out_hbm.at[c].at[sl])

  return k(ego2, colg, rowg, valg)


def _transform(side, ego, W_gc, b_gc, W_bi, b_bi):
  R = 2000

  def body(s_ref, e_ref, wg_ref, bg_ref, wb_ref, bb_ref, eout_ref, nout_ref):
    s_lo = s_ref[0]
    s_hi = s_ref[1]
    e = e_ref[...]
    e_lo = e[:, :32]
    e_hi = e[:, 32:]
    wg = wg_ref[...]
    wb = wb_ref[...]
    sum_e = (jnp.dot(s_lo, wg[:32], preferred_element_type=jnp.float32) +
             jnp.dot(s_hi, wg[32:], preferred_element_type=jnp.float32) +
             bg_ref[...])
    bi_e = (jnp.dot(e_lo * s_lo, wb[:32], preferred_element_type=jnp.float32) +
            jnp.dot(e_hi * s_hi, wb[32:], preferred_element_type=jnp.float32) +
            bb_ref[...])
    act = sum_e + bi_e
    act = jnp.where(act >= 0, act, 0.2 * act)
    eout_ref[...] = act
    ss = jnp.sum(act * act, axis=1, keepdims=True)
    nout_ref[...] = act / jnp.maximum(jnp.sqrt(ss), 1e-12)

  return pl.pallas_call(
      body,
      grid=(_N_NODES // R,),
      in_specs=[
          pl.BlockSpec((2, R, 32), lambda r: (0, r, 0)),
          pl.BlockSpec((R, _D), lambda r: (r, 0)),
          pl.BlockSpec((_D, _D), lambda r: (0, 0)),
          pl.BlockSpec((1, _D), lambda r: (0, 0)),
          pl.BlockSpec((_D, _D), lambda r: (0, 0)),
          pl.BlockSpec((1, _D), lambda r: (0, 0)),
      ],
      out_specs=[
          pl.BlockSpec((R, _D), lambda r: (r, 0)),
          pl.BlockSpec((R, _D), lambda r: (r, 0)),
      ],
      out_shape=[jax.ShapeDtypeStruct((_N_NODES, _D), jnp.float32)] * 2,
      compiler_params=pltpu.CompilerParams(
          dimension_semantics=("arbitrary",)),
  )(side, ego, W_gc, b_gc, W_bi, b_bi)


def _final(t0, t1, t2, t3, u2, i2, j2):
  mesh = plsc.VectorSubcoreMesh(core_axis_name="c", subcore_axis_name="s",
                                num_cores=2, num_subcores=_NT)
  out_sd = jax.ShapeDtypeStruct((_BATCH, _D), jnp.float32)

  @functools.partial(
      pl.kernel,
      out_type=[out_sd] * 12,
      mesh=mesh,
      scratch_types=[
          pltpu.VMEM((128,), jnp.int32),
          pltpu.VMEM((128, _D), jnp.float32),
          pltpu.SemaphoreType.DMA,
      ],
      compiler_params=pltpu.CompilerParams(use_tc_tiling_on_sc=False),
  )
  def k(t0_hbm, t1_hbm, t2_hbm, t3_hbm, u_hbm, i_hbm, j_hbm, *rest):
    outs = rest[:12]
    idxv, buf, sem = rest[12:]
    c = lax.axis_index("c")
    s = lax.axis_index("s")
    w = s * 2 + c
    for p, (src, off) in enumerate(((u_hbm, 0), (i_hbm, _N_USERS),
                                    (j_hbm, _N_USERS))):
      pltpu.sync_copy(src.at[w], idxv)
      if off:
        for t in range(8):
          sl = pl.ds(t * 16, 16)
          idxv[sl] = idxv[sl] + off
      for tt, tbl in enumerate((t0_hbm, t1_hbm, t2_hbm, t3_hbm)):
        pltpu.async_copy(tbl.at[idxv], buf, sem).wait()
        pltpu.sync_copy(buf, outs[p * 4 + tt].at[pl.ds(w * 128, 128)])

  return k(t0, t1, t2, t3, u2, i2, j2)


def kernel(u, i, j, adj_indices, adj_values, user_emb, item_emb,
           W_gc_0, b_gc_0, W_bi_0, b_bi_0,
           W_gc_1, b_gc_1, W_bi_1, b_bi_1,
           W_gc_2, b_gc_2, W_bi_2, b_bi_2):
  rows_d = adj_indices[0].astype(jnp.int32)
  cols = adj_indices[1].astype(jnp.int32)
  vals = adj_values
  pad = _EP - _E
  colg = jnp.concatenate([cols, jnp.zeros((pad,), jnp.int32)]).reshape(
      _EP // 64, 64)
  rowg = jnp.concatenate([rows_d, jnp.zeros((pad,), jnp.int32)]).reshape(
      _EP // 128, 128)
  valg = jnp.concatenate([vals, jnp.zeros((pad,), jnp.float32)])

  ego = jnp.concatenate([user_emb, item_emb], axis=0)
  tables = [ego]
  for (Wg, bg, Wb, bb) in ((W_gc_0, b_gc_0, W_bi_0, b_bi_0),
                           (W_gc_1, b_gc_1, W_bi_1, b_bi_1),
                           (W_gc_2, b_gc_2, W_bi_2, b_bi_2)):
    side = _segsum(ego.reshape(2 * _N_NODES, 32), colg, rowg, valg)
    ego, nk = _transform(side, ego, Wg, bg, Wb, bb)
    tables.append(nk)

  u2 = u.astype(jnp.int32).reshape(32, 128)
  i2 = i.astype(jnp.int32).reshape(32, 128)
  j2 = j.astype(jnp.int32).reshape(32, 128)
  parts = _final(tables[0], tables[1], tables[2], tables[3], u2, i2, j2)
  return (jnp.concatenate(parts[0:4], axis=1),
          jnp.concatenate(parts[4:8], axis=1),
          jnp.concatenate(parts[8:12], axis=1))
